# initial kernel scaffold (unmeasured)
import jax
import jax.numpy as jnp
from jax import lax
from jax.experimental import pallas as pl
from jax.experimental.pallas import tpu as pltpu

N_DEV = 8
SQ_BLK = 256
D_MODEL = 1024
HEADS = 8
DH = 128
SKV = 4096
WIN = 512
SCALE = 0.08838834764831843


def _blk(ref, i):
    return ref[pl.ds(i, 1)][0]


def kernel(x, Wq, K_ext, V_ext, Wo):
    d = lax.axis_index("i")
    hs = d * HEADS

    x_bf = x[0].astype(jnp.bfloat16)
    wq_bf = Wq.astype(jnp.bfloat16)
    wo_bf = Wo.astype(jnp.bfloat16)
    k_mine = lax.dynamic_slice_in_dim(K_ext[0], hs, HEADS, axis=1)
    k_mine = jnp.transpose(k_mine, (1, 0, 2)).astype(jnp.bfloat16)
    v_mine = lax.dynamic_slice_in_dim(V_ext[0], hs, HEADS, axis=1)
    v_mine = jnp.transpose(v_mine, (1, 0, 2)).astype(jnp.bfloat16)

    def body(x_ref, wq_ref, k_ref, v_ref, wo_ref, out_ref,
             xfull, partials, rs_recv,
             ag_send_sems, ag_recv_sems, rs_send_sems, rs_recv_sems):
        my = lax.axis_index("i")
        left = lax.rem(my + N_DEV - 1, N_DEV)
        right = lax.rem(my + 1, N_DEV)

        barrier = pltpu.get_barrier_semaphore()
        for nbr in (left, right):
            pl.semaphore_signal(barrier, inc=1, device_id=(nbr,),
                                device_id_type=pl.DeviceIdType.MESH)
        pl.semaphore_wait(barrier, 2)

        xfull[pl.ds(my, 1)] = x_ref[...][None]

        for h in range(N_DEV - 1):
            b = lax.rem(my - h + N_DEV, N_DEV)
            rdma = pltpu.make_async_remote_copy(
                src_ref=xfull.at[b],
                dst_ref=xfull.at[b],
                send_sem=ag_send_sems.at[h],
                recv_sem=ag_recv_sems.at[h],
                device_id=(right,),
                device_id_type=pl.DeviceIdType.MESH,
            )
            rdma.start()
            rdma.wait()

        def compute_block(h, _):
            b = lax.rem(my - h + N_DEV, N_DEV)
            xb = _blk(xfull, b)
            q = jnp.dot(xb, wq_ref[...],
                        preferred_element_type=jnp.float32)
            qb = (q * SCALE).astype(jnp.bfloat16)
            start = jnp.clip(b * SQ_BLK - 128, 0, SKV - WIN)
            ctx_parts = []
            for hh in range(HEADS):
                qh = qb[:, hh * DH:(hh + 1) * DH]
                kh = k_ref[hh, pl.ds(start, WIN), :]
                s = lax.dot_general(qh, kh, (((1,), (1,)), ((), ())),
                                    preferred_element_type=jnp.float32)
                qi = b * SQ_BLK + lax.broadcasted_iota(
                    jnp.int32, (SQ_BLK, WIN), 0)
                ki = start + lax.broadcasted_iota(
                    jnp.int32, (SQ_BLK, WIN), 1)
                s = jnp.where(jnp.abs(qi - ki) <= 128, s, -1e9)
                m = jnp.max(s, axis=1, keepdims=True)
                w = jnp.exp(s - m)
                w = w / jnp.sum(w, axis=1, keepdims=True)
                vh = v_ref[hh, pl.ds(start, WIN), :]
                ctx_parts.append(jnp.dot(w.astype(jnp.bfloat16), vh,
                                         preferred_element_type=jnp.float32))
            ctx = jnp.concatenate(ctx_parts, axis=1).astype(jnp.bfloat16)
            partial = jnp.dot(ctx, wo_ref[...],
                              preferred_element_type=jnp.float32)
            partials[pl.ds(b, 1)] = partial.astype(jnp.bfloat16)[None]
            return 0

        lax.fori_loop(0, N_DEV, compute_block, 0)

        for s_ in range(N_DEV - 1):
            bsend = lax.rem(my - 1 - s_ + 2 * N_DEV, N_DEV)
            if s_ > 0:
                acc = (_blk(partials, bsend).astype(jnp.float32)
                       + rs_recv[s_ - 1].astype(jnp.float32))
                partials[pl.ds(bsend, 1)] = acc.astype(jnp.bfloat16)[None]
            rdma = pltpu.make_async_remote_copy(
                src_ref=partials.at[bsend],
                dst_ref=rs_recv.at[s_],
                send_sem=rs_send_sems.at[s_],
                recv_sem=rs_recv_sems.at[s_],
                device_id=(right,),
                device_id_type=pl.DeviceIdType.MESH,
            )
            rdma.start()
            rdma.wait()

        out_ref[0] = (_blk(partials, my).astype(jnp.float32)
                      + rs_recv[N_DEV - 2].astype(jnp.float32))

    out = pl.pallas_call(
        body,
        out_shape=jax.ShapeDtypeStruct((1, SQ_BLK, D_MODEL), jnp.float32),
        in_specs=[pl.BlockSpec(memory_space=pltpu.VMEM)] * 5,
        out_specs=pl.BlockSpec(memory_space=pltpu.VMEM),
        scratch_shapes=[
            pltpu.VMEM((N_DEV, SQ_BLK, D_MODEL), jnp.bfloat16),
            pltpu.VMEM((N_DEV, SQ_BLK, D_MODEL), jnp.bfloat16),
            pltpu.VMEM((N_DEV - 1, SQ_BLK, D_MODEL), jnp.bfloat16),
            pltpu.SemaphoreType.DMA((N_DEV - 1,)),
            pltpu.SemaphoreType.DMA((N_DEV - 1,)),
            pltpu.SemaphoreType.DMA((N_DEV - 1,)),
            pltpu.SemaphoreType.DMA((N_DEV - 1,)),
        ],
        compiler_params=pltpu.CompilerParams(collective_id=0),
    )(x_bf, wq_bf, k_mine, v_mine, wo_bf)
    return out


# baseline (device time: 192300 ns/iter reference)
import jax
import jax.numpy as jnp
from jax import lax
from jax.experimental import pallas as pl
from jax.experimental.pallas import tpu as pltpu

N_DEV = 8
SQ_BLK = 256
D_MODEL = 1024
HEADS = 8
DH = 128
SKV = 4096
WIN = 512
SCALE = 0.08838834764831843


def _blk(ref, i):
    return ref[pl.ds(i, 1)][0]


def kernel(x, Wq, K_ext, V_ext, Wo):
    d = lax.axis_index("i")
    hs = d * HEADS

    x_bf = x[0].astype(jnp.bfloat16)
    wq_bf = Wq.astype(jnp.bfloat16)
    wo_bf = Wo.astype(jnp.bfloat16)
    k_mine = lax.dynamic_slice_in_dim(K_ext[0], hs, HEADS, axis=1)
    k_mine = jnp.transpose(k_mine, (1, 0, 2)).astype(jnp.bfloat16)
    v_mine = lax.dynamic_slice_in_dim(V_ext[0], hs, HEADS, axis=1)
    v_mine = jnp.transpose(v_mine, (1, 0, 2)).astype(jnp.bfloat16)

    def body(x_ref, wq_ref, k_ref, v_ref, wo_ref, out_ref,
             xfull, partials, rs_recv,
             ag_send_sems, ag_recv_sems, rs_send_sems, rs_recv_sems):
        my = lax.axis_index("i")
        left = lax.rem(my + N_DEV - 1, N_DEV)
        right = lax.rem(my + 1, N_DEV)

        barrier = pltpu.get_barrier_semaphore()
        for nbr in (left, right):
            pl.semaphore_signal(barrier, inc=1, device_id=(nbr,),
                                device_id_type=pl.DeviceIdType.MESH)
        pl.semaphore_wait(barrier, 2)

        xfull[pl.ds(my, 1)] = x_ref[...][None]

        for h in range(N_DEV - 1):
            b = lax.rem(my - h + N_DEV, N_DEV)
            rdma = pltpu.make_async_remote_copy(
                src_ref=xfull.at[b],
                dst_ref=xfull.at[b],
                send_sem=ag_send_sems.at[h],
                recv_sem=ag_recv_sems.at[h],
                device_id=(right,),
                device_id_type=pl.DeviceIdType.MESH,
            )
            rdma.start()
            rdma.wait()

        def compute_block(h, _):
            b = lax.rem(my - h + N_DEV, N_DEV)
            xb = _blk(xfull, b)
            q = jnp.dot(xb, wq_ref[...],
                        preferred_element_type=jnp.float32)
            qb = (q * SCALE).astype(jnp.bfloat16)
            start = jnp.clip(b * SQ_BLK - 128, 0, SKV - WIN)
            start = pl.multiple_of(start, 128)
            ctx_parts = []
            for hh in range(HEADS):
                qh = qb[:, hh * DH:(hh + 1) * DH]
                kh = k_ref[hh, pl.ds(start, WIN), :]
                s = lax.dot_general(qh, kh, (((1,), (1,)), ((), ())),
                                    preferred_element_type=jnp.float32)
                qi = b * SQ_BLK + lax.broadcasted_iota(
                    jnp.int32, (SQ_BLK, WIN), 0)
                ki = start + lax.broadcasted_iota(
                    jnp.int32, (SQ_BLK, WIN), 1)
                s = jnp.where(jnp.abs(qi - ki) <= 128, s, -1e9)
                m = jnp.max(s, axis=1, keepdims=True)
                w = jnp.exp(s - m)
                w = w / jnp.sum(w, axis=1, keepdims=True)
                vh = v_ref[hh, pl.ds(start, WIN), :]
                ctx_parts.append(jnp.dot(w.astype(jnp.bfloat16), vh,
                                         preferred_element_type=jnp.float32))
            ctx = jnp.concatenate(ctx_parts, axis=1).astype(jnp.bfloat16)
            partial = jnp.dot(ctx, wo_ref[...],
                              preferred_element_type=jnp.float32)
            partials[pl.ds(b, 1)] = partial.astype(jnp.bfloat16)[None]
            return 0

        lax.fori_loop(0, N_DEV, compute_block, 0)

        for s_ in range(N_DEV - 1):
            bsend = lax.rem(my - 1 - s_ + 2 * N_DEV, N_DEV)
            if s_ > 0:
                acc = (_blk(partials, bsend).astype(jnp.float32)
                       + rs_recv[s_ - 1].astype(jnp.float32))
                partials[pl.ds(bsend, 1)] = acc.astype(jnp.bfloat16)[None]
            rdma = pltpu.make_async_remote_copy(
                src_ref=partials.at[bsend],
                dst_ref=rs_recv.at[s_],
                send_sem=rs_send_sems.at[s_],
                recv_sem=rs_recv_sems.at[s_],
                device_id=(right,),
                device_id_type=pl.DeviceIdType.MESH,
            )
            rdma.start()
            rdma.wait()

        out_ref[0] = (_blk(partials, my).astype(jnp.float32)
                      + rs_recv[N_DEV - 2].astype(jnp.float32))

    out = pl.pallas_call(
        body,
        out_shape=jax.ShapeDtypeStruct((1, SQ_BLK, D_MODEL), jnp.float32),
        in_specs=[pl.BlockSpec(memory_space=pltpu.VMEM)] * 5,
        out_specs=pl.BlockSpec(memory_space=pltpu.VMEM),
        scratch_shapes=[
            pltpu.VMEM((N_DEV, SQ_BLK, D_MODEL), jnp.bfloat16),
            pltpu.VMEM((N_DEV, SQ_BLK, D_MODEL), jnp.bfloat16),
            pltpu.VMEM((N_DEV - 1, SQ_BLK, D_MODEL), jnp.bfloat16),
            pltpu.SemaphoreType.DMA((N_DEV - 1,)),
            pltpu.SemaphoreType.DMA((N_DEV - 1,)),
            pltpu.SemaphoreType.DMA((N_DEV - 1,)),
            pltpu.SemaphoreType.DMA((N_DEV - 1,)),
        ],
        compiler_params=pltpu.CompilerParams(collective_id=0),
    )(x_bf, wq_bf, k_mine, v_mine, wo_bf)
    return out


# device time: 118684 ns/iter; 1.6203x vs baseline; 1.6203x over previous
import jax
import jax.numpy as jnp
from jax import lax
from jax.experimental import pallas as pl
from jax.experimental.pallas import tpu as pltpu

N_DEV = 8
SQ_BLK = 256
RH = 128
D_MODEL = 1024
HEADS = 8
DH = 128
SKV = 4096
WIN = 384
SCALE = 0.08838834764831843


def kernel(x, Wq, K_ext, V_ext, Wo):
    d = lax.axis_index("i")
    hs = d * HEADS

    x_bf = x[0].astype(jnp.bfloat16)
    wq_bf = Wq.astype(jnp.bfloat16)
    wo_bf = Wo.astype(jnp.bfloat16)
    k_mine = lax.dynamic_slice_in_dim(K_ext[0], hs, HEADS, axis=1)
    k_mine = jnp.transpose(k_mine, (1, 0, 2)).astype(jnp.bfloat16)
    v_mine = lax.dynamic_slice_in_dim(V_ext[0], hs, HEADS, axis=1)
    v_mine = jnp.transpose(v_mine, (1, 0, 2)).astype(jnp.bfloat16)

    def body(x_ref, wq_ref, k_ref, v_ref, wo_ref, out_ref,
             xf_r, xf_l, pa_r, pa_l, rr_r, rr_l,
             ag_ss_r, ag_rs_r, rs_ss_r, rs_rs_r,
             ag_ss_l, ag_rs_l, rs_ss_l, rs_rs_l):
        my = lax.axis_index("i")
        left = lax.rem(my + N_DEV - 1, N_DEV)
        right = lax.rem(my + 1, N_DEV)

        barrier = pltpu.get_barrier_semaphore()
        for nbr in (left, right):
            pl.semaphore_signal(barrier, inc=1, device_id=(nbr,),
                                device_id_type=pl.DeviceIdType.MESH)
        pl.semaphore_wait(barrier, 2)

        xf_r[pl.ds(my, 1)] = x_ref[:RH][None]
        xf_l[pl.ds(my, 1)] = x_ref[RH:][None]

        rings = (
            (+1, 0, xf_r, pa_r, rr_r, ag_ss_r, ag_rs_r, rs_ss_r, rs_rs_r,
             right),
            (-1, RH, xf_l, pa_l, rr_l, ag_ss_l, ag_rs_l, rs_ss_l, rs_rs_l,
             left),
        )

        def compute_half(b, off, xf, extra):
            xb = xf[pl.ds(b, 1)][0]
            q = jnp.dot(xb, wq_ref[...],
                        preferred_element_type=jnp.float32)
            qb = (q * SCALE).astype(jnp.bfloat16)
            start = jnp.clip(b * SQ_BLK + off - 128, 0, SKV - WIN)
            start = pl.multiple_of(start, 128)
            ctx_parts = []
            for hh in range(HEADS):
                qh = qb[:, hh * DH:(hh + 1) * DH]
                kh = k_ref[hh, pl.ds(start, WIN), :]
                s = lax.dot_general(qh, kh, (((1,), (1,)), ((), ())),
                                    preferred_element_type=jnp.float32)
                qi = b * SQ_BLK + off + lax.broadcasted_iota(
                    jnp.int32, (RH, WIN), 0)
                ki = start + lax.broadcasted_iota(jnp.int32, (RH, WIN), 1)
                s = jnp.where(jnp.abs(qi - ki) <= 128, s, -1e9)
                m = jnp.max(s, axis=1, keepdims=True)
                w = jnp.exp(s - m)
                w = w / jnp.sum(w, axis=1, keepdims=True)
                vh = v_ref[hh, pl.ds(start, WIN), :]
                ctx_parts.append(jnp.dot(w.astype(jnp.bfloat16), vh,
                                         preferred_element_type=jnp.float32))
            ctx = jnp.concatenate(ctx_parts, axis=1).astype(jnp.bfloat16)
            partial = jnp.dot(ctx, wo_ref[...],
                              preferred_element_type=jnp.float32)
            if extra is not None:
                partial = partial + extra.astype(jnp.float32)
            return partial.astype(jnp.bfloat16)

        ag_sends = []
        rs_sends = []
        for h in range(N_DEV):
            if h < N_DEV - 1:
                for (dr, off, xf, pa, rr, ag_ss, ag_rs, rs_ss, rs_rs,
                     tgt) in rings:
                    b = lax.rem(my - dr * h + 2 * N_DEV, N_DEV)
                    rdma = pltpu.make_async_remote_copy(
                        src_ref=xf.at[b], dst_ref=xf.at[b],
                        send_sem=ag_ss.at[h], recv_sem=ag_rs.at[h],
                        device_id=(tgt,),
                        device_id_type=pl.DeviceIdType.MESH,
                    )
                    rdma.start()
                    ag_sends.append(rdma)

            for dr, off, xf, pa, rr, ag_ss, ag_rs, rs_ss, rs_rs, tgt in rings:
                b = lax.rem(my - dr * h + 2 * N_DEV, N_DEV)
                extra = None
                if h >= 2:
                    pltpu.make_async_remote_copy(
                        src_ref=rr.at[h - 2], dst_ref=rr.at[h - 2],
                        send_sem=rs_ss.at[h - 2], recv_sem=rs_rs.at[h - 2],
                        device_id=(tgt,),
                        device_id_type=pl.DeviceIdType.MESH,
                    ).wait_recv()
                    extra = rr[h - 2]
                chunk = compute_half(b, off, xf, extra)
                pa[pl.ds(b, 1)] = chunk[None]
                if h >= 1:
                    rdma = pltpu.make_async_remote_copy(
                        src_ref=pa.at[b], dst_ref=rr.at[h - 1],
                        send_sem=rs_ss.at[h - 1], recv_sem=rs_rs.at[h - 1],
                        device_id=(tgt,),
                        device_id_type=pl.DeviceIdType.MESH,
                    )
                    rdma.start()
                    rs_sends.append(rdma)

            if h < N_DEV - 1:
                for (dr, off, xf, pa, rr, ag_ss, ag_rs, rs_ss, rs_rs,
                     tgt) in rings:
                    b_next = lax.rem(my - dr * (h + 1) + 2 * N_DEV, N_DEV)
                    pltpu.make_async_remote_copy(
                        src_ref=xf.at[b_next], dst_ref=xf.at[b_next],
                        send_sem=ag_ss.at[h], recv_sem=ag_rs.at[h],
                        device_id=(tgt,),
                        device_id_type=pl.DeviceIdType.MESH,
                    ).wait_recv()

        outs = []
        for dr, off, xf, pa, rr, ag_ss, ag_rs, rs_ss, rs_rs, tgt in rings:
            pltpu.make_async_remote_copy(
                src_ref=rr.at[N_DEV - 2], dst_ref=rr.at[N_DEV - 2],
                send_sem=rs_ss.at[N_DEV - 2], recv_sem=rs_rs.at[N_DEV - 2],
                device_id=(tgt,), device_id_type=pl.DeviceIdType.MESH,
            ).wait_recv()
            outs.append((pa[pl.ds(my, 1)][0].astype(jnp.float32)
                         + rr[N_DEV - 2].astype(jnp.float32)))
        out_ref[0] = jnp.concatenate(outs, axis=0)

        for rdma in ag_sends + rs_sends:
            rdma.wait_send()

    out = pl.pallas_call(
        body,
        out_shape=jax.ShapeDtypeStruct((1, SQ_BLK, D_MODEL), jnp.float32),
        in_specs=[pl.BlockSpec(memory_space=pltpu.VMEM)] * 5,
        out_specs=pl.BlockSpec(memory_space=pltpu.VMEM),
        scratch_shapes=[
            pltpu.VMEM((N_DEV, RH, D_MODEL), jnp.bfloat16),
            pltpu.VMEM((N_DEV, RH, D_MODEL), jnp.bfloat16),
            pltpu.VMEM((N_DEV, RH, D_MODEL), jnp.bfloat16),
            pltpu.VMEM((N_DEV, RH, D_MODEL), jnp.bfloat16),
            pltpu.VMEM((N_DEV - 1, RH, D_MODEL), jnp.bfloat16),
            pltpu.VMEM((N_DEV - 1, RH, D_MODEL), jnp.bfloat16),
        ] + [pltpu.SemaphoreType.DMA((N_DEV - 1,))] * 8,
        compiler_params=pltpu.CompilerParams(collective_id=0),
    )(x_bf, wq_bf, k_mine, v_mine, wo_bf)
    return out


# device time: 98423 ns/iter; 1.9538x vs baseline; 1.2059x over previous
import jax
import jax.numpy as jnp
from jax import lax
from jax.experimental import pallas as pl
from jax.experimental.pallas import tpu as pltpu

N_DEV = 8
SQ_BLK = 256
RH = 128
D_MODEL = 1024
HEADS = 8
DH = 128
SKV = 4096
WIN = 384
SCALE = 0.08838834764831843


def kernel(x, Wq, K_ext, V_ext, Wo):
    d = lax.axis_index("i")
    hs = d * HEADS

    x_bf = x[0].astype(jnp.bfloat16)
    wq_bf = Wq.astype(jnp.bfloat16)
    wo_bf = Wo.astype(jnp.bfloat16)
    k_mine = lax.dynamic_slice_in_dim(K_ext[0], hs, HEADS, axis=1)
    k_mine = k_mine.reshape(SKV, HEADS * DH).astype(jnp.bfloat16)
    v_mine = lax.dynamic_slice_in_dim(V_ext[0], hs, HEADS, axis=1)
    v_mine = v_mine.reshape(SKV, HEADS * DH).astype(jnp.bfloat16)

    def body(x_ref, wq_ref, k_ref, v_ref, wo_ref, out_ref,
             xf_r, xf_l, pa_r, pa_l, rr_r, rr_l,
             ag_ss_r, ag_rs_r, rs_ss_r, rs_rs_r,
             ag_ss_l, ag_rs_l, rs_ss_l, rs_rs_l):
        my = lax.axis_index("i")
        left = lax.rem(my + N_DEV - 1, N_DEV)
        right = lax.rem(my + 1, N_DEV)

        barrier = pltpu.get_barrier_semaphore()
        for nbr in (left, right):
            pl.semaphore_signal(barrier, inc=1, device_id=(nbr,),
                                device_id_type=pl.DeviceIdType.MESH)
        pl.semaphore_wait(barrier, 2)

        xf_r[pl.ds(my, 1)] = x_ref[:RH][None]
        xf_l[pl.ds(my, 1)] = x_ref[RH:][None]

        rings = (
            (+1, 0, xf_r, pa_r, rr_r, ag_ss_r, ag_rs_r, rs_ss_r, rs_rs_r,
             right),
            (-1, RH, xf_l, pa_l, rr_l, ag_ss_l, ag_rs_l, rs_ss_l, rs_rs_l,
             left),
        )

        def compute_half(b, off, xf, extra):
            xb = xf[pl.ds(b, 1)][0]
            q = jnp.dot(xb, wq_ref[...],
                        preferred_element_type=jnp.float32)
            qb = (q * SCALE).astype(jnp.bfloat16)
            start = jnp.clip(b * SQ_BLK + off - 128, 0, SKV - WIN)
            start = pl.multiple_of(start, 128)
            ctx_parts = []
            for hh in range(HEADS):
                qh = qb[:, hh * DH:(hh + 1) * DH]
                kh = k_ref[pl.ds(start, WIN), hh * DH:(hh + 1) * DH]
                s = lax.dot_general(qh, kh, (((1,), (1,)), ((), ())),
                                    preferred_element_type=jnp.float32)
                qi = b * SQ_BLK + off + lax.broadcasted_iota(
                    jnp.int32, (RH, WIN), 0)
                ki = start + lax.broadcasted_iota(jnp.int32, (RH, WIN), 1)
                s = jnp.where(jnp.abs(qi - ki) <= 128, s, -1e9)
                m = jnp.max(s, axis=1, keepdims=True)
                w = jnp.exp(s - m)
                w = w / jnp.sum(w, axis=1, keepdims=True)
                vh = v_ref[pl.ds(start, WIN), hh * DH:(hh + 1) * DH]
                ctx_parts.append(jnp.dot(w.astype(jnp.bfloat16), vh,
                                         preferred_element_type=jnp.float32))
            ctx = jnp.concatenate(ctx_parts, axis=1).astype(jnp.bfloat16)
            partial = jnp.dot(ctx, wo_ref[...],
                              preferred_element_type=jnp.float32)
            if extra is not None:
                partial = partial + extra.astype(jnp.float32)
            return partial.astype(jnp.bfloat16)

        ag_sends = []
        rs_sends = []

        def rs_step(ring, s):
            dr, off, xf, pa, rr, ag_ss, ag_rs, rs_ss, rs_rs, tgt = ring
            bs = lax.rem(my - dr * (s + 1) + 2 * N_DEV, N_DEV)
            if s >= 1:
                pltpu.make_async_remote_copy(
                    src_ref=rr.at[s - 1], dst_ref=rr.at[s - 1],
                    send_sem=rs_ss.at[s - 1], recv_sem=rs_rs.at[s - 1],
                    device_id=(tgt,), device_id_type=pl.DeviceIdType.MESH,
                ).wait_recv()
                acc = (pa[pl.ds(bs, 1)][0].astype(jnp.float32)
                       + rr[s - 1].astype(jnp.float32))
                pa[pl.ds(bs, 1)] = acc.astype(jnp.bfloat16)[None]
            rdma = pltpu.make_async_remote_copy(
                src_ref=pa.at[bs], dst_ref=rr.at[s],
                send_sem=rs_ss.at[s], recv_sem=rs_rs.at[s],
                device_id=(tgt,), device_id_type=pl.DeviceIdType.MESH,
            )
            rdma.start()
            rs_sends.append(rdma)

        for h in range(N_DEV):
            if h < N_DEV - 1:
                for ring in rings:
                    dr, off, xf = ring[0], ring[1], ring[2]
                    ag_ss, ag_rs, tgt = ring[5], ring[6], ring[9]
                    b = lax.rem(my - dr * h + 2 * N_DEV, N_DEV)
                    rdma = pltpu.make_async_remote_copy(
                        src_ref=xf.at[b], dst_ref=xf.at[b],
                        send_sem=ag_ss.at[h], recv_sem=ag_rs.at[h],
                        device_id=(tgt,),
                        device_id_type=pl.DeviceIdType.MESH,
                    )
                    rdma.start()
                    ag_sends.append(rdma)

            if h >= 2:
                for ring in rings:
                    rs_step(ring, h - 2)

            for dr, off, xf, pa, rr, ag_ss, ag_rs, rs_ss, rs_rs, tgt in rings:
                b = lax.rem(my - dr * h + 2 * N_DEV, N_DEV)
                chunk = compute_half(b, off, xf, None)
                pa[pl.ds(b, 1)] = chunk[None]

            if h < N_DEV - 1:
                for (dr, off, xf, pa, rr, ag_ss, ag_rs, rs_ss, rs_rs,
                     tgt) in rings:
                    b_next = lax.rem(my - dr * (h + 1) + 2 * N_DEV, N_DEV)
                    pltpu.make_async_remote_copy(
                        src_ref=xf.at[b_next], dst_ref=xf.at[b_next],
                        send_sem=ag_ss.at[h], recv_sem=ag_rs.at[h],
                        device_id=(tgt,),
                        device_id_type=pl.DeviceIdType.MESH,
                    ).wait_recv()

        for ring in rings:
            rs_step(ring, N_DEV - 2)

        outs = []
        for dr, off, xf, pa, rr, ag_ss, ag_rs, rs_ss, rs_rs, tgt in rings:
            pltpu.make_async_remote_copy(
                src_ref=rr.at[N_DEV - 2], dst_ref=rr.at[N_DEV - 2],
                send_sem=rs_ss.at[N_DEV - 2], recv_sem=rs_rs.at[N_DEV - 2],
                device_id=(tgt,), device_id_type=pl.DeviceIdType.MESH,
            ).wait_recv()
            outs.append((pa[pl.ds(my, 1)][0].astype(jnp.float32)
                         + rr[N_DEV - 2].astype(jnp.float32)))
        out_ref[0] = jnp.concatenate(outs, axis=0)

        for rdma in ag_sends + rs_sends:
            rdma.wait_send()

    out = pl.pallas_call(
        body,
        out_shape=jax.ShapeDtypeStruct((1, SQ_BLK, D_MODEL), jnp.float32),
        in_specs=[pl.BlockSpec(memory_space=pltpu.VMEM)] * 5,
        out_specs=pl.BlockSpec(memory_space=pltpu.VMEM),
        scratch_shapes=[
            pltpu.VMEM((N_DEV, RH, D_MODEL), jnp.bfloat16),
            pltpu.VMEM((N_DEV, RH, D_MODEL), jnp.bfloat16),
            pltpu.VMEM((N_DEV, RH, D_MODEL), jnp.bfloat16),
            pltpu.VMEM((N_DEV, RH, D_MODEL), jnp.bfloat16),
            pltpu.VMEM((N_DEV - 1, RH, D_MODEL), jnp.bfloat16),
            pltpu.VMEM((N_DEV - 1, RH, D_MODEL), jnp.bfloat16),
        ] + [pltpu.SemaphoreType.DMA((N_DEV - 1,))] * 8,
        compiler_params=pltpu.CompilerParams(collective_id=0),
    )(x_bf, wq_bf, k_mine, v_mine, wo_bf)
    return out
